# Initial kernel scaffold; baseline (speedup 1.0000x reference)
import jax
import jax.numpy as jnp
from jax.experimental import pallas as pl


def kernel(x, table, gamma, beta):
    raise NotImplementedError("write your pallas kernel here")


# speedup vs baseline: 10.3667x; 10.3667x over previous
import functools
import math

import numpy as np
import jax
import jax.numpy as jnp
from jax import lax
from jax.experimental import pallas as pl
from jax.experimental.pallas import tpu as pltpu
from jax.experimental.pallas import tpu_sc as plsc

VOCAB = 259
D = 128
B = 4096
S = 200
VP = 264
NC, NS = 2, 16
NW = NC * NS
TOK = B * S
TPW = TOK // NW
CH = 128
NCH = TPW // CH
NBUF = 4

LANES = 128
ROWS = TOK // LANES
XBLK = 640

SBLK = 8


def _pe_np():
    position = np.arange(0, S, dtype=np.float32)[:, None]
    div_term = np.exp(
        np.arange(0, D, 2, dtype=np.float32) * (-math.log(10000.0) / D))
    pe = np.zeros((S, D), dtype=np.float32)
    pe[:, 0::2] = np.sin(position * div_term)
    pe[:, 1::2] = np.cos(position * div_term)
    return pe


_PE = _pe_np()


def _fused_body(tab_ref, pe_ref, g_ref, b_ref, f_ref):
    h = tab_ref[...][None, :, :] + pe_ref[...][:, None, :]
    m = jnp.mean(h, axis=-1, keepdims=True)
    r = h - m
    v = jnp.mean(r * r, axis=-1, keepdims=True)
    f_ref[...] = r * lax.rsqrt(v + 1e-5) * g_ref[...] + b_ref[...]


def _fused_table(table_pad, pe, gamma2, beta2):
    return pl.pallas_call(
        _fused_body,
        grid=(S // SBLK,),
        in_specs=[
            pl.BlockSpec((VP, D), lambda i: (0, 0)),
            pl.BlockSpec((SBLK, D), lambda i: (i, 0)),
            pl.BlockSpec((1, D), lambda i: (0, 0)),
            pl.BlockSpec((1, D), lambda i: (0, 0)),
        ],
        out_specs=pl.BlockSpec((SBLK, VP, D), lambda i: (i, 0, 0)),
        out_shape=jax.ShapeDtypeStruct((S, VP, D), jnp.float32),
    )(table_pad, pe, gamma2, beta2)


def _idx_body(x_ref, idx_ref):
    pid = pl.program_id(0)
    r = lax.broadcasted_iota(jnp.int32, (XBLK, LANES), 0)
    c = lax.broadcasted_iota(jnp.int32, (XBLK, LANES), 1)
    t = (pid * XBLK + r) * LANES + c
    s = t % S
    idx_ref[...] = x_ref[...] + s * VP


def _idx_kernel(xflat):
    return pl.pallas_call(
        _idx_body,
        grid=(ROWS // XBLK,),
        in_specs=[pl.BlockSpec((XBLK, LANES), lambda i: (i, 0))],
        out_specs=pl.BlockSpec((XBLK, LANES), lambda i: (i, 0)),
        out_shape=jax.ShapeDtypeStruct((ROWS, LANES), jnp.int32),
    )(xflat)


def _sc_gather(f_flat, idx):
    mesh = plsc.VectorSubcoreMesh(core_axis_name="c", subcore_axis_name="s")

    @functools.partial(
        pl.kernel,
        out_type=jax.ShapeDtypeStruct((TOK, D), jnp.float32),
        mesh=mesh,
        scratch_types=[
            pltpu.VMEM((NBUF, CH), jnp.int32),
            pltpu.VMEM((NBUF, CH, D), jnp.float32),
            pltpu.SemaphoreType.DMA,
            pltpu.SemaphoreType.DMA,
            pltpu.SemaphoreType.DMA,
            pltpu.SemaphoreType.DMA,
        ],
    )
    def k(f_hbm, idx_hbm, out_hbm, idx_v, rows_v, s0, s1, s2, s3):
        sems = [s0, s1, s2, s3]
        wid = lax.axis_index("s") * NC + lax.axis_index("c")
        base = wid * TPW

        def fetch(j, slot):
            off = pl.multiple_of(base + j * CH, CH)
            pltpu.sync_copy(idx_hbm.at[pl.ds(off, CH)], idx_v.at[slot])
            pltpu.async_copy(f_hbm.at[idx_v.at[slot]], rows_v.at[slot],
                             sems[slot])

        def drain_write(j, slot):
            pltpu.make_async_copy(f_hbm.at[idx_v.at[slot]], rows_v.at[slot],
                                  sems[slot]).wait()
            off = pl.multiple_of(base + j * CH, CH)
            pltpu.sync_copy(rows_v.at[slot], out_hbm.at[pl.ds(off, CH)])

        for p in range(NBUF):
            fetch(p, p)

        def body(i, carry):
            j0 = i * NBUF
            for p in range(NBUF):
                drain_write(j0 + p, p)
                fetch(j0 + p + NBUF, p)
            return carry

        lax.fori_loop(0, (NCH - NBUF) // NBUF, body, 0)

        j0 = NCH - NBUF
        for p in range(NBUF):
            drain_write(j0 + p, p)

    return k(f_flat, idx)


def kernel(x, table, gamma, beta):
    x = x.astype(jnp.int32)
    table_pad = jnp.zeros((VP, D), table.dtype).at[:VOCAB].set(table)
    pe = jnp.asarray(_PE)
    f = _fused_table(table_pad, pe, gamma.reshape(1, D), beta.reshape(1, D))
    f_flat = f.reshape(S * VP, D)
    idx = _idx_kernel(x.reshape(ROWS, LANES)).reshape(TOK)
    out = _sc_gather(f_flat, idx)
    return out.reshape(B, S, D)


# speedup vs baseline: 10.4415x; 1.0072x over previous
import functools
import math

import numpy as np
import jax
import jax.numpy as jnp
from jax import lax
from jax.experimental import pallas as pl
from jax.experimental.pallas import tpu as pltpu
from jax.experimental.pallas import tpu_sc as plsc

VOCAB = 259
D = 128
B = 4096
S = 200
VP = 264
NC, NS = 2, 16
NW = NC * NS
TOK = B * S
TPW = TOK // NW
CH = 128
NCH = TPW // CH
NBUF = 4

LANES = 128
ROWS = TOK // LANES
XBLK = 640

SBLK = 8


def _pe_np():
    position = np.arange(0, S, dtype=np.float32)[:, None]
    div_term = np.exp(
        np.arange(0, D, 2, dtype=np.float32) * (-math.log(10000.0) / D))
    pe = np.zeros((S, D), dtype=np.float32)
    pe[:, 0::2] = np.sin(position * div_term)
    pe[:, 1::2] = np.cos(position * div_term)
    return pe


_PE = _pe_np()


def _fused_body(tab_ref, pe_ref, g_ref, b_ref, f_ref):
    h = tab_ref[...][None, :, :] + pe_ref[...][:, None, :]
    m = jnp.mean(h, axis=-1, keepdims=True)
    r = h - m
    v = jnp.mean(r * r, axis=-1, keepdims=True)
    f_ref[...] = r * lax.rsqrt(v + 1e-5) * g_ref[...] + b_ref[...]


def _fused_table(table_pad, pe, gamma2, beta2):
    return pl.pallas_call(
        _fused_body,
        grid=(S // SBLK,),
        in_specs=[
            pl.BlockSpec((VP, D), lambda i: (0, 0)),
            pl.BlockSpec((SBLK, D), lambda i: (i, 0)),
            pl.BlockSpec((1, D), lambda i: (0, 0)),
            pl.BlockSpec((1, D), lambda i: (0, 0)),
        ],
        out_specs=pl.BlockSpec((SBLK, VP, D), lambda i: (i, 0, 0)),
        out_shape=jax.ShapeDtypeStruct((S, VP, D), jnp.float32),
    )(table_pad, pe, gamma2, beta2)


def _idx_body(x_ref, idx_ref):
    pid = pl.program_id(0)
    r = lax.broadcasted_iota(jnp.int32, (XBLK, LANES), 0)
    c = lax.broadcasted_iota(jnp.int32, (XBLK, LANES), 1)
    t = (pid * XBLK + r) * LANES + c
    s = t % S
    idx_ref[...] = x_ref[...] + s * VP


def _idx_kernel(xflat):
    return pl.pallas_call(
        _idx_body,
        grid=(ROWS // XBLK,),
        in_specs=[pl.BlockSpec((XBLK, LANES), lambda i: (i, 0))],
        out_specs=pl.BlockSpec((XBLK, LANES), lambda i: (i, 0)),
        out_shape=jax.ShapeDtypeStruct((ROWS, LANES), jnp.int32),
    )(xflat)


def _sc_gather(f_flat, idx3):
    mesh = plsc.VectorSubcoreMesh(core_axis_name="c", subcore_axis_name="s")

    @functools.partial(
        pl.kernel,
        out_type=jax.ShapeDtypeStruct((TOK, D), jnp.float32),
        mesh=mesh,
        scratch_types=[
            pltpu.VMEM((NCH, CH), jnp.int32),
            pltpu.VMEM((NBUF, CH, D), jnp.float32),
            pltpu.SemaphoreType.DMA,
            pltpu.SemaphoreType.DMA,
            pltpu.SemaphoreType.DMA,
            pltpu.SemaphoreType.DMA,
            pltpu.SemaphoreType.DMA,
            pltpu.SemaphoreType.DMA,
            pltpu.SemaphoreType.DMA,
            pltpu.SemaphoreType.DMA,
        ],
    )
    def k(f_hbm, idx_hbm, out_hbm, idx_v, rows_v,
          g0, g1, g2, g3, w0, w1, w2, w3):
        gsems = [g0, g1, g2, g3]
        wsems = [w0, w1, w2, w3]
        wid = lax.axis_index("s") * NC + lax.axis_index("c")
        base = wid * TPW

        pltpu.sync_copy(idx_hbm.at[wid], idx_v)

        def gather(j, slot):
            pltpu.async_copy(f_hbm.at[idx_v.at[j]], rows_v.at[slot],
                             gsems[slot])

        def fetch(j, slot):
            pltpu.make_async_copy(
                rows_v.at[slot],
                out_hbm.at[pl.ds(pl.multiple_of(base, CH), CH)],
                wsems[slot]).wait()
            gather(j, slot)

        def drain(j, slot):
            pltpu.make_async_copy(f_hbm.at[idx_v.at[j]], rows_v.at[slot],
                                  gsems[slot]).wait()
            off = pl.multiple_of(base + j * CH, CH)
            pltpu.async_copy(rows_v.at[slot], out_hbm.at[pl.ds(off, CH)],
                             wsems[slot])

        for p in range(NBUF - 1):
            gather(p, p)
        drain(0, 0)
        gather(NBUF - 1, NBUF - 1)

        def body(i, carry):
            k0 = i * NBUF + 1
            for p in range(NBUF):
                k = k0 + p
                drain(k, (1 + p) % NBUF)
                fetch(k + NBUF - 1, p)
            return carry

        lax.fori_loop(0, (NCH - NBUF) // NBUF, body, 0)

        j0 = NCH - NBUF
        for p in range(1, NBUF):
            drain(j0 + p, (j0 + p) % NBUF)

        for p in range(NBUF):
            pltpu.make_async_copy(
                rows_v.at[p],
                out_hbm.at[pl.ds(pl.multiple_of(base, CH), CH)],
                wsems[p]).wait()

    return k(f_flat, idx3)


def kernel(x, table, gamma, beta):
    x = x.astype(jnp.int32)
    table_pad = jnp.zeros((VP, D), table.dtype).at[:VOCAB].set(table)
    pe = jnp.asarray(_PE)
    f = _fused_table(table_pad, pe, gamma.reshape(1, D), beta.reshape(1, D))
    f_flat = f.reshape(S * VP, D)
    idx = _idx_kernel(x.reshape(ROWS, LANES)).reshape(NW, NCH, CH)
    out = _sc_gather(f_flat, idx)
    return out.reshape(B, S, D)


# speedup vs baseline: 10.6222x; 1.0173x over previous
import functools
import math

import numpy as np
import jax
import jax.numpy as jnp
from jax import lax
from jax.experimental import pallas as pl
from jax.experimental.pallas import tpu as pltpu
from jax.experimental.pallas import tpu_sc as plsc

VOCAB = 259
D = 128
B = 4096
S = 200
VP = 264
NC, NS = 2, 16
NW = NC * NS
TOK = B * S
TPW = TOK // NW
CH = 128
NCH = TPW // CH
NBUF = 4

LANES = 128
ROWS = TOK // LANES
XBLK = 256

SBLK = 8


def _pe_np():
    position = np.arange(0, S, dtype=np.float32)[:, None]
    div_term = np.exp(
        np.arange(0, D, 2, dtype=np.float32) * (-math.log(10000.0) / D))
    pe = np.zeros((S, D), dtype=np.float32)
    pe[:, 0::2] = np.sin(position * div_term)
    pe[:, 1::2] = np.cos(position * div_term)
    return pe


_PE = _pe_np()


def _prep_body(tab_ref, pe_ref, g_ref, b_ref, x_ref, f_ref, idx_ref):
    h = tab_ref[...][None, :, :] + pe_ref[...][:, None, :]
    m = jnp.mean(h, axis=-1, keepdims=True)
    r = h - m
    v = jnp.mean(r * r, axis=-1, keepdims=True)
    f_ref[...] = r * lax.rsqrt(v + 1e-5) * g_ref[...] + b_ref[...]
    pid = pl.program_id(0)
    rr = lax.broadcasted_iota(jnp.int32, (XBLK, LANES), 0)
    cc = lax.broadcasted_iota(jnp.int32, (XBLK, LANES), 1)
    t = (pid * XBLK + rr) * LANES + cc
    idx_ref[...] = x_ref[...] + (t % S) * VP


def _prep_kernel(table_pad, pe, gamma2, beta2, xflat):
    return pl.pallas_call(
        _prep_body,
        grid=(S // SBLK,),
        in_specs=[
            pl.BlockSpec((VP, D), lambda i: (0, 0)),
            pl.BlockSpec((SBLK, D), lambda i: (i, 0)),
            pl.BlockSpec((1, D), lambda i: (0, 0)),
            pl.BlockSpec((1, D), lambda i: (0, 0)),
            pl.BlockSpec((XBLK, LANES), lambda i: (i, 0)),
        ],
        out_specs=[
            pl.BlockSpec((SBLK, VP, D), lambda i: (i, 0, 0)),
            pl.BlockSpec((XBLK, LANES), lambda i: (i, 0)),
        ],
        out_shape=[
            jax.ShapeDtypeStruct((S, VP, D), jnp.float32),
            jax.ShapeDtypeStruct((ROWS, LANES), jnp.int32),
        ],
    )(table_pad, pe, gamma2, beta2, xflat)


def _sc_gather(f_flat, idx3):
    mesh = plsc.VectorSubcoreMesh(core_axis_name="c", subcore_axis_name="s")

    @functools.partial(
        pl.kernel,
        out_type=jax.ShapeDtypeStruct((TOK, D), jnp.float32),
        mesh=mesh,
        scratch_types=[
            pltpu.VMEM((NCH, CH), jnp.int32),
            pltpu.VMEM((NBUF, CH, D), jnp.float32),
            pltpu.SemaphoreType.DMA,
            pltpu.SemaphoreType.DMA,
            pltpu.SemaphoreType.DMA,
            pltpu.SemaphoreType.DMA,
            pltpu.SemaphoreType.DMA,
            pltpu.SemaphoreType.DMA,
            pltpu.SemaphoreType.DMA,
            pltpu.SemaphoreType.DMA,
        ],
    )
    def k(f_hbm, idx_hbm, out_hbm, idx_v, rows_v,
          g0, g1, g2, g3, w0, w1, w2, w3):
        gsems = [g0, g1, g2, g3]
        wsems = [w0, w1, w2, w3]
        wid = lax.axis_index("s") * NC + lax.axis_index("c")
        base = wid * TPW

        pltpu.sync_copy(idx_hbm.at[wid], idx_v)

        def gather(j, slot):
            pltpu.async_copy(f_hbm.at[idx_v.at[j]], rows_v.at[slot],
                             gsems[slot])

        def fetch(j, slot):
            pltpu.make_async_copy(
                rows_v.at[slot],
                out_hbm.at[pl.ds(pl.multiple_of(base, CH), CH)],
                wsems[slot]).wait()
            gather(j, slot)

        def drain(j, slot):
            pltpu.make_async_copy(f_hbm.at[idx_v.at[j]], rows_v.at[slot],
                                  gsems[slot]).wait()
            off = pl.multiple_of(base + j * CH, CH)
            pltpu.async_copy(rows_v.at[slot], out_hbm.at[pl.ds(off, CH)],
                             wsems[slot])

        for p in range(NBUF - 1):
            gather(p, p)
        drain(0, 0)
        gather(NBUF - 1, NBUF - 1)

        def body(i, carry):
            k0 = i * NBUF + 1
            for p in range(NBUF):
                k = k0 + p
                drain(k, (1 + p) % NBUF)
                fetch(k + NBUF - 1, p)
            return carry

        lax.fori_loop(0, (NCH - NBUF) // NBUF, body, 0)

        j0 = NCH - NBUF
        for p in range(1, NBUF):
            drain(j0 + p, (j0 + p) % NBUF)

        for p in range(NBUF):
            pltpu.make_async_copy(
                rows_v.at[p],
                out_hbm.at[pl.ds(pl.multiple_of(base, CH), CH)],
                wsems[p]).wait()

    return k(f_flat, idx3)


def kernel(x, table, gamma, beta):
    x = x.astype(jnp.int32)
    table_pad = jnp.zeros((VP, D), table.dtype).at[:VOCAB].set(table)
    pe = jnp.asarray(_PE)
    f, idx = _prep_kernel(table_pad, pe, gamma.reshape(1, D),
                          beta.reshape(1, D), x.reshape(ROWS, LANES))
    f_flat = f.reshape(S * VP, D)
    out = _sc_gather(f_flat, idx.reshape(NW, NCH, CH))
    return out.reshape(B, S, D)


# speedup vs baseline: 10.6321x; 1.0009x over previous
import functools
import math

import numpy as np
import jax
import jax.numpy as jnp
from jax import lax
from jax.experimental import pallas as pl
from jax.experimental.pallas import tpu as pltpu
from jax.experimental.pallas import tpu_sc as plsc

VOCAB = 259
D = 128
B = 4096
S = 200
VP = 264
NC, NS = 2, 16
NW = NC * NS
TOK = B * S
TPW = TOK // NW
CH = 128
NCH = TPW // CH
NBUF = 5

LANES = 128
ROWS = TOK // LANES
XBLK = 256

SBLK = 8


def _pe_np():
    position = np.arange(0, S, dtype=np.float32)[:, None]
    div_term = np.exp(
        np.arange(0, D, 2, dtype=np.float32) * (-math.log(10000.0) / D))
    pe = np.zeros((S, D), dtype=np.float32)
    pe[:, 0::2] = np.sin(position * div_term)
    pe[:, 1::2] = np.cos(position * div_term)
    return pe


_PE = _pe_np()


def _prep_body(tab_ref, pe_ref, g_ref, b_ref, x_ref, f_ref, idx_ref):
    h = tab_ref[...][None, :, :] + pe_ref[...][:, None, :]
    m = jnp.mean(h, axis=-1, keepdims=True)
    r = h - m
    v = jnp.mean(r * r, axis=-1, keepdims=True)
    f_ref[...] = r * lax.rsqrt(v + 1e-5) * g_ref[...] + b_ref[...]
    pid = pl.program_id(0)
    rr = lax.broadcasted_iota(jnp.int32, (XBLK, LANES), 0)
    cc = lax.broadcasted_iota(jnp.int32, (XBLK, LANES), 1)
    t = (pid * XBLK + rr) * LANES + cc
    idx_ref[...] = x_ref[...] + (t % S) * VP


def _prep_kernel(table_pad, pe, gamma2, beta2, xflat):
    return pl.pallas_call(
        _prep_body,
        grid=(S // SBLK,),
        in_specs=[
            pl.BlockSpec((VP, D), lambda i: (0, 0)),
            pl.BlockSpec((SBLK, D), lambda i: (i, 0)),
            pl.BlockSpec((1, D), lambda i: (0, 0)),
            pl.BlockSpec((1, D), lambda i: (0, 0)),
            pl.BlockSpec((XBLK, LANES), lambda i: (i, 0)),
        ],
        out_specs=[
            pl.BlockSpec((SBLK, VP, D), lambda i: (i, 0, 0)),
            pl.BlockSpec((XBLK, LANES), lambda i: (i, 0)),
        ],
        out_shape=[
            jax.ShapeDtypeStruct((S, VP, D), jnp.float32),
            jax.ShapeDtypeStruct((ROWS, LANES), jnp.int32),
        ],
    )(table_pad, pe, gamma2, beta2, xflat)


def _sc_gather(f_flat, idx3):
    mesh = plsc.VectorSubcoreMesh(core_axis_name="c", subcore_axis_name="s")

    @functools.partial(
        pl.kernel,
        out_type=jax.ShapeDtypeStruct((TOK, D), jnp.float32),
        mesh=mesh,
        scratch_types=[
            pltpu.VMEM((NCH, CH), jnp.int32),
            pltpu.VMEM((NBUF, CH, D), jnp.float32),
            pltpu.SemaphoreType.DMA,
            pltpu.SemaphoreType.DMA,
            pltpu.SemaphoreType.DMA,
            pltpu.SemaphoreType.DMA,
            pltpu.SemaphoreType.DMA,
            pltpu.SemaphoreType.DMA,
            pltpu.SemaphoreType.DMA,
            pltpu.SemaphoreType.DMA,
            pltpu.SemaphoreType.DMA,
            pltpu.SemaphoreType.DMA,
        ],
    )
    def k(f_hbm, idx_hbm, out_hbm, idx_v, rows_v,
          g0, g1, g2, g3, g4, w0, w1, w2, w3, w4):
        gsems = [g0, g1, g2, g3, g4]
        wsems = [w0, w1, w2, w3, w4]
        wid = lax.axis_index("s") * NC + lax.axis_index("c")
        base = wid * TPW

        pltpu.sync_copy(idx_hbm.at[wid], idx_v)

        def gather(j, slot):
            pltpu.async_copy(f_hbm.at[idx_v.at[j]], rows_v.at[slot],
                             gsems[slot])

        def fetch(j, slot):
            pltpu.make_async_copy(
                rows_v.at[slot],
                out_hbm.at[pl.ds(pl.multiple_of(base, CH), CH)],
                wsems[slot]).wait()
            gather(j, slot)

        def drain(j, slot):
            pltpu.make_async_copy(f_hbm.at[idx_v.at[j]], rows_v.at[slot],
                                  gsems[slot]).wait()
            off = pl.multiple_of(base + j * CH, CH)
            pltpu.async_copy(rows_v.at[slot], out_hbm.at[pl.ds(off, CH)],
                             wsems[slot])

        for p in range(NBUF - 1):
            gather(p, p)
        drain(0, 0)
        gather(NBUF - 1, NBUF - 1)

        def body(i, carry):
            k0 = i * NBUF + 1
            for p in range(NBUF):
                k = k0 + p
                drain(k, (1 + p) % NBUF)
                fetch(k + NBUF - 1, p)
            return carry

        lax.fori_loop(0, (NCH - NBUF) // NBUF, body, 0)

        j0 = NCH - NBUF
        for p in range(1, NBUF):
            drain(j0 + p, (j0 + p) % NBUF)

        for p in range(NBUF):
            pltpu.make_async_copy(
                rows_v.at[p],
                out_hbm.at[pl.ds(pl.multiple_of(base, CH), CH)],
                wsems[p]).wait()

    return k(f_flat, idx3)


def kernel(x, table, gamma, beta):
    x = x.astype(jnp.int32)
    table_pad = jnp.zeros((VP, D), table.dtype).at[:VOCAB].set(table)
    pe = jnp.asarray(_PE)
    f, idx = _prep_kernel(table_pad, pe, gamma.reshape(1, D),
                          beta.reshape(1, D), x.reshape(ROWS, LANES))
    f_flat = f.reshape(S * VP, D)
    out = _sc_gather(f_flat, idx.reshape(NW, NCH, CH))
    return out.reshape(B, S, D)


# speedup vs baseline: 10.7948x; 1.0153x over previous
import functools
import math

import numpy as np
import jax
import jax.numpy as jnp
from jax import lax
from jax.experimental import pallas as pl
from jax.experimental.pallas import tpu as pltpu
from jax.experimental.pallas import tpu_sc as plsc

VOCAB = 259
D = 128
B = 4096
S = 200
VP = 264
NC, NS = 2, 16
NW = NC * NS
TOK = B * S
TPW = TOK // NW
CH = 128
NCH = TPW // CH
NBUF = 5

LANES = 128
ROWS = TOK // LANES
XBLK = 1280

SBLK = 40


def _pe_np():
    position = np.arange(0, S, dtype=np.float32)[:, None]
    div_term = np.exp(
        np.arange(0, D, 2, dtype=np.float32) * (-math.log(10000.0) / D))
    pe = np.zeros((S, D), dtype=np.float32)
    pe[:, 0::2] = np.sin(position * div_term)
    pe[:, 1::2] = np.cos(position * div_term)
    return pe


_PE = _pe_np()


def _prep_body(tab_ref, pe_ref, g_ref, b_ref, x_ref, f_ref, idx_ref):
    h = tab_ref[...][None, :, :] + pe_ref[...][:, None, :]
    m = jnp.mean(h, axis=-1, keepdims=True)
    r = h - m
    v = jnp.mean(r * r, axis=-1, keepdims=True)
    f_ref[...] = r * lax.rsqrt(v + 1e-5) * g_ref[...] + b_ref[...]
    pid = pl.program_id(0)
    rr = lax.broadcasted_iota(jnp.int32, (XBLK, LANES), 0)
    cc = lax.broadcasted_iota(jnp.int32, (XBLK, LANES), 1)
    t = (pid * XBLK + rr) * LANES + cc
    idx_ref[...] = x_ref[...] + (t % S) * VP


def _prep_kernel(table_pad, pe, gamma2, beta2, xflat):
    return pl.pallas_call(
        _prep_body,
        grid=(S // SBLK,),
        in_specs=[
            pl.BlockSpec((VP, D), lambda i: (0, 0)),
            pl.BlockSpec((SBLK, D), lambda i: (i, 0)),
            pl.BlockSpec((1, D), lambda i: (0, 0)),
            pl.BlockSpec((1, D), lambda i: (0, 0)),
            pl.BlockSpec((XBLK, LANES), lambda i: (i, 0)),
        ],
        out_specs=[
            pl.BlockSpec((SBLK, VP, D), lambda i: (i, 0, 0)),
            pl.BlockSpec((XBLK, LANES), lambda i: (i, 0)),
        ],
        out_shape=[
            jax.ShapeDtypeStruct((S, VP, D), jnp.float32),
            jax.ShapeDtypeStruct((ROWS, LANES), jnp.int32),
        ],
    )(table_pad, pe, gamma2, beta2, xflat)


def _sc_gather(f_flat, idx3):
    mesh = plsc.VectorSubcoreMesh(core_axis_name="c", subcore_axis_name="s")

    @functools.partial(
        pl.kernel,
        out_type=jax.ShapeDtypeStruct((TOK, D), jnp.float32),
        mesh=mesh,
        scratch_types=[
            pltpu.VMEM((NCH, CH), jnp.int32),
            pltpu.VMEM((NBUF, CH, D), jnp.float32),
            pltpu.SemaphoreType.DMA,
            pltpu.SemaphoreType.DMA,
            pltpu.SemaphoreType.DMA,
            pltpu.SemaphoreType.DMA,
            pltpu.SemaphoreType.DMA,
            pltpu.SemaphoreType.DMA,
            pltpu.SemaphoreType.DMA,
            pltpu.SemaphoreType.DMA,
            pltpu.SemaphoreType.DMA,
            pltpu.SemaphoreType.DMA,
        ],
    )
    def k(f_hbm, idx_hbm, out_hbm, idx_v, rows_v,
          g0, g1, g2, g3, g4, w0, w1, w2, w3, w4):
        gsems = [g0, g1, g2, g3, g4]
        wsems = [w0, w1, w2, w3, w4]
        wid = lax.axis_index("s") * NC + lax.axis_index("c")
        base = wid * TPW

        pltpu.sync_copy(idx_hbm.at[wid], idx_v)

        def gather(j, slot):
            pltpu.async_copy(f_hbm.at[idx_v.at[j]], rows_v.at[slot],
                             gsems[slot])

        def fetch(j, slot):
            pltpu.make_async_copy(
                rows_v.at[slot],
                out_hbm.at[pl.ds(pl.multiple_of(base, CH), CH)],
                wsems[slot]).wait()
            gather(j, slot)

        def drain(j, slot):
            pltpu.make_async_copy(f_hbm.at[idx_v.at[j]], rows_v.at[slot],
                                  gsems[slot]).wait()
            off = pl.multiple_of(base + j * CH, CH)
            pltpu.async_copy(rows_v.at[slot], out_hbm.at[pl.ds(off, CH)],
                             wsems[slot])

        for p in range(NBUF - 1):
            gather(p, p)
        drain(0, 0)
        gather(NBUF - 1, NBUF - 1)

        def body(i, carry):
            k0 = i * NBUF + 1
            for p in range(NBUF):
                k = k0 + p
                drain(k, (1 + p) % NBUF)
                fetch(k + NBUF - 1, p)
            return carry

        lax.fori_loop(0, (NCH - NBUF) // NBUF, body, 0)

        j0 = NCH - NBUF
        for p in range(1, NBUF):
            drain(j0 + p, (j0 + p) % NBUF)

        for p in range(NBUF):
            pltpu.make_async_copy(
                rows_v.at[p],
                out_hbm.at[pl.ds(pl.multiple_of(base, CH), CH)],
                wsems[p]).wait()

    return k(f_flat, idx3)


def kernel(x, table, gamma, beta):
    x = x.astype(jnp.int32)
    table_pad = jnp.zeros((VP, D), table.dtype).at[:VOCAB].set(table)
    pe = jnp.asarray(_PE)
    f, idx = _prep_kernel(table_pad, pe, gamma.reshape(1, D),
                          beta.reshape(1, D), x.reshape(ROWS, LANES))
    f_flat = f.reshape(S * VP, D)
    out = _sc_gather(f_flat, idx.reshape(NW, NCH, CH))
    return out.reshape(B, S, D)
